# baseline (speedup 1.0000x reference)
import jax
import jax.numpy as jnp
from jax import lax
from jax.experimental import pallas as pl
from jax.experimental.pallas import tpu as pltpu
from jax.experimental.pallas import tpu_sc as plsc

PROMPT_NUM = 50
HIDDEN_SIZE = 1024
BATCH = 1024

_K = 16
_ROWS = _K * PROMPT_NUM
_NBULK = BATCH // _K
_MROWS = BATCH * PROMPT_NUM // 128


def _tc_body(table_v, emb_hbm, mask_hbm, staged, ones_v, sem, mask_sem):
    staged[...] = jnp.zeros((_ROWS, HIDDEN_SIZE), jnp.float32)
    ones_v[...] = jnp.ones((_MROWS, 128), jnp.float32)
    bulk = [
        pltpu.make_async_copy(staged, emb_hbm.at[pl.ds(j * _ROWS, _ROWS)], sem)
        for j in range(_NBULK)
    ]
    mask_h = pltpu.make_async_copy(ones_v, mask_hbm, mask_sem)
    mask_h.start()
    for h in bulk:
        h.start()
    for h in bulk:
        h.wait()
    mask_h.wait()


def _tc_broadcast(prompt_table):
    return pl.pallas_call(
        _tc_body,
        out_shape=(
            jax.ShapeDtypeStruct((BATCH * PROMPT_NUM, HIDDEN_SIZE), jnp.float32),
            jax.ShapeDtypeStruct((_MROWS, 128), jnp.float32),
        ),
        in_specs=[pl.BlockSpec(memory_space=pltpu.VMEM)],
        out_specs=(
            pl.BlockSpec(memory_space=pl.ANY),
            pl.BlockSpec(memory_space=pl.ANY),
        ),
        scratch_shapes=[
            pltpu.VMEM((_ROWS, HIDDEN_SIZE), jnp.float32),
            pltpu.VMEM((_MROWS, 128), jnp.float32),
            pltpu.SemaphoreType.DMA,
            pltpu.SemaphoreType.DMA,
        ],
    )(prompt_table)


def kernel(batch_size, prompt_table):
    emb_flat, mask_flat = _tc_broadcast(prompt_table)
    emb = emb_flat.reshape(BATCH, PROMPT_NUM, HIDDEN_SIZE)
    mask = mask_flat.reshape(BATCH, PROMPT_NUM)
    return emb, mask


# speedup vs baseline: 1.6188x; 1.6188x over previous
import jax
import jax.numpy as jnp
from jax import lax
from jax.experimental import pallas as pl
from jax.experimental.pallas import tpu as pltpu
from jax.experimental.pallas import tpu_sc as plsc

PROMPT_NUM = 50
HIDDEN_SIZE = 1024
BATCH = 1024

_NSTREAM = 8
_K = 16
_PER = BATCH // _NSTREAM
_NDESC = _PER // _K


def _tc_body(table_v, emb_hbm, mask_hbm, *rest):
    staged = rest[:_NSTREAM]
    ones_v = rest[_NSTREAM]
    sems = rest[_NSTREAM + 1]
    mask_sem = rest[_NSTREAM + 2]
    for s in staged:
        s[...] = jnp.broadcast_to(
            table_v[...][None], (_K, PROMPT_NUM, HIDDEN_SIZE)
        )
    ones_v[...] = jnp.ones((BATCH, PROMPT_NUM), jnp.float32)
    handles = [
        pltpu.make_async_copy(
            staged[i],
            emb_hbm.at[pl.ds(i * _PER + j * _K, _K)],
            sems.at[i],
        )
        for i in range(_NSTREAM)
        for j in range(_NDESC)
    ]
    mask_h = pltpu.make_async_copy(ones_v, mask_hbm, mask_sem)
    mask_h.start()
    for h in handles:
        h.start()
    for h in handles:
        h.wait()
    mask_h.wait()


def _tc_broadcast(prompt_table):
    return pl.pallas_call(
        _tc_body,
        out_shape=(
            jax.ShapeDtypeStruct((BATCH, PROMPT_NUM, HIDDEN_SIZE), jnp.float32),
            jax.ShapeDtypeStruct((BATCH, PROMPT_NUM), jnp.float32),
        ),
        in_specs=[pl.BlockSpec(memory_space=pltpu.VMEM)],
        out_specs=(
            pl.BlockSpec(memory_space=pl.ANY),
            pl.BlockSpec(memory_space=pl.ANY),
        ),
        scratch_shapes=(
            [pltpu.VMEM((_K, PROMPT_NUM, HIDDEN_SIZE), jnp.float32)
             for _ in range(_NSTREAM)]
            + [
                pltpu.VMEM((BATCH, PROMPT_NUM), jnp.float32),
                pltpu.SemaphoreType.DMA((_NSTREAM,)),
                pltpu.SemaphoreType.DMA,
            ]
        ),
    )(prompt_table)


def kernel(batch_size, prompt_table):
    emb, mask = _tc_broadcast(prompt_table)
    return emb, mask


# speedup vs baseline: 5.7094x; 3.5270x over previous
import jax
import jax.numpy as jnp
from jax import lax
from jax.experimental import pallas as pl
from jax.experimental.pallas import tpu as pltpu
from jax.experimental.pallas import tpu_sc as plsc

PROMPT_NUM = 50
HIDDEN_SIZE = 1024
BATCH = 1024

_NBUF = 4


def _tc_body(table_v, emb_hbm, mask_hbm, staged, ones_v, sems, mask_sem):
    ones_v[...] = jnp.ones((PROMPT_NUM, BATCH), jnp.float32)
    mask_h = pltpu.make_async_copy(ones_v, mask_hbm, mask_sem)
    mask_h.start()
    handles = []
    for p in range(PROMPT_NUM):
        s = p % _NBUF
        if p >= _NBUF:
            handles[p - _NBUF].wait()
        staged[s, ...] = jnp.broadcast_to(
            table_v[pl.ds(p, 1), :], (BATCH, HIDDEN_SIZE)
        )
        h = pltpu.make_async_copy(staged.at[s], emb_hbm.at[p], sems.at[s])
        h.start()
        handles.append(h)
    for p in range(PROMPT_NUM - _NBUF, PROMPT_NUM):
        handles[p].wait()
    mask_h.wait()


def _tc_broadcast(prompt_table):
    return pl.pallas_call(
        _tc_body,
        out_shape=(
            jax.ShapeDtypeStruct((PROMPT_NUM, BATCH, HIDDEN_SIZE), jnp.float32),
            jax.ShapeDtypeStruct((PROMPT_NUM, BATCH), jnp.float32),
        ),
        in_specs=[pl.BlockSpec(memory_space=pltpu.VMEM)],
        out_specs=(
            pl.BlockSpec(memory_space=pl.ANY),
            pl.BlockSpec(memory_space=pl.ANY),
        ),
        scratch_shapes=[
            pltpu.VMEM((_NBUF, BATCH, HIDDEN_SIZE), jnp.float32),
            pltpu.VMEM((PROMPT_NUM, BATCH), jnp.float32),
            pltpu.SemaphoreType.DMA((_NBUF,)),
            pltpu.SemaphoreType.DMA,
        ],
    )(prompt_table)


def kernel(batch_size, prompt_table):
    emb_t, mask_t = _tc_broadcast(prompt_table)
    emb = jnp.transpose(emb_t, (1, 0, 2))
    mask = jnp.transpose(mask_t, (1, 0))
    return emb, mask
